# Initial kernel scaffold; baseline (speedup 1.0000x reference)
import jax
import jax.numpy as jnp
from jax.experimental import pallas as pl


def kernel(point_clouds, alphas, sigma):
    raise NotImplementedError("write your pallas kernel here")


# speedup vs baseline: 79.1374x; 79.1374x over previous
import functools

import jax
import jax.numpy as jnp
from jax import lax
from jax.experimental import pallas as pl
from jax.experimental.pallas import tpu as pltpu
from jax.experimental.pallas import tpu_sc as plsc

_N_WEIGHTS = 4
_DIM = 16
_J = 3
_B = 4
_N = 4096

_NUM_CORES = 2
_NUM_SUBCORES = 16
_CLOUDS_PER_CORE = _B // _NUM_CORES
_ROWS_PER_SUBCORE = _N // _NUM_SUBCORES
_OUT_COLS = _N_WEIGHTS * (_J + 1) * _DIM


def _sc_kernel(pc_hbm, al_hbm, out_hbm, rows_v, part_v, all_v, al_v, row_v,
               shared):
    c = lax.axis_index("c")
    s = lax.axis_index("s")
    zero = jnp.zeros((_DIM,), jnp.float32)

    for lp in range(_CLOUDS_PER_CORE):
        p = c * _CLOUDS_PER_CORE + lp
        pltpu.sync_copy(
            pc_hbm.at[p, pl.ds(s * _ROWS_PER_SUBCORE, _ROWS_PER_SUBCORE), :],
            rows_v,
        )

        def body(i, acc):
            a0, a1, a2, a3 = acc
            a0 = a0 + rows_v[4 * i]
            a1 = a1 + rows_v[4 * i + 1]
            a2 = a2 + rows_v[4 * i + 2]
            a3 = a3 + rows_v[4 * i + 3]
            return (a0, a1, a2, a3)

        accs = lax.fori_loop(0, _ROWS_PER_SUBCORE // 4, body,
                             (zero, zero, zero, zero))
        part_v[pl.ds(lp * _DIM, _DIM)] = (accs[0] + accs[1]) + (accs[2] + accs[3])

    pltpu.sync_copy(part_v,
                    shared.at[pl.ds(s * _CLOUDS_PER_CORE * _DIM,
                                    _CLOUDS_PER_CORE * _DIM)])
    plsc.subcore_barrier()

    @pl.when(s == 0)
    def _():
        pltpu.sync_copy(shared, all_v)
        pltpu.sync_copy(al_hbm, al_v)
        for lp in range(_CLOUDS_PER_CORE):
            tot = zero
            for sid in range(_NUM_SUBCORES):
                tot = tot + all_v[pl.ds((sid * _CLOUDS_PER_CORE + lp) * _DIM,
                                        _DIM)]
            for w in range(_N_WEIGHTS):
                prod = al_v[w] * tot
                for j in range(_J + 1):
                    row_v[pl.ds((w * (_J + 1) + j) * _DIM, _DIM)] = prod
            pltpu.sync_copy(row_v, out_hbm.at[c * _CLOUDS_PER_CORE + lp])


@jax.jit
def _hiponet_pooled(point_clouds, alphas):
    mesh = plsc.VectorSubcoreMesh(core_axis_name="c", subcore_axis_name="s",
                                  num_cores=_NUM_CORES,
                                  num_subcores=_NUM_SUBCORES)
    return pl.kernel(
        _sc_kernel,
        out_type=jax.ShapeDtypeStruct((_B, _OUT_COLS), jnp.float32),
        mesh=mesh,
        scratch_types=[
            pltpu.VMEM((_ROWS_PER_SUBCORE, _DIM), jnp.float32),
            pltpu.VMEM((_CLOUDS_PER_CORE * _DIM,), jnp.float32),
            pltpu.VMEM((_NUM_SUBCORES * _CLOUDS_PER_CORE * _DIM,),
                       jnp.float32),
            pltpu.VMEM((_N_WEIGHTS, _DIM), jnp.float32),
            pltpu.VMEM((_OUT_COLS,), jnp.float32),
            pltpu.VMEM_SHARED((_NUM_SUBCORES * _CLOUDS_PER_CORE * _DIM,),
                              jnp.float32),
        ],
        name="hiponet_pooled_sc",
    )(point_clouds, alphas)


def kernel(point_clouds, alphas, sigma):
    del sigma
    return _hiponet_pooled(point_clouds, alphas)


# speedup vs baseline: 82.3550x; 1.0407x over previous
import functools

import jax
import jax.numpy as jnp
from jax import lax
from jax.experimental import pallas as pl
from jax.experimental.pallas import tpu as pltpu
from jax.experimental.pallas import tpu_sc as plsc

_N_WEIGHTS = 4
_DIM = 16
_J = 3
_B = 4
_N = 4096

_NUM_CORES = 2
_NUM_SUBCORES = 16
_CLOUDS_PER_CORE = _B // _NUM_CORES
_SUBCORES_PER_CLOUD = _NUM_SUBCORES // _CLOUDS_PER_CORE
_ROWS_PER_SUBCORE = _N // _SUBCORES_PER_CLOUD
_OUT_COLS = _N_WEIGHTS * (_J + 1) * _DIM


def _sc_kernel(pc_hbm, al_hbm, out_hbm, rows_v, part_v, all_v, al_v, row_v,
               shared):
    c = lax.axis_index("c")
    s = lax.axis_index("s")
    zero = jnp.zeros((_DIM,), jnp.float32)

    p = c * _CLOUDS_PER_CORE + s // _SUBCORES_PER_CLOUD
    base = (s % _SUBCORES_PER_CLOUD) * _ROWS_PER_SUBCORE
    pltpu.sync_copy(pc_hbm.at[p, pl.ds(base, _ROWS_PER_SUBCORE), :], rows_v)

    def body(i, acc):
        a0, a1, a2, a3, a4, a5, a6, a7 = acc
        a0 = a0 + rows_v[8 * i]
        a1 = a1 + rows_v[8 * i + 1]
        a2 = a2 + rows_v[8 * i + 2]
        a3 = a3 + rows_v[8 * i + 3]
        a4 = a4 + rows_v[8 * i + 4]
        a5 = a5 + rows_v[8 * i + 5]
        a6 = a6 + rows_v[8 * i + 6]
        a7 = a7 + rows_v[8 * i + 7]
        return (a0, a1, a2, a3, a4, a5, a6, a7)

    accs = lax.fori_loop(0, _ROWS_PER_SUBCORE // 8, body, (zero,) * 8)
    part_v[:] = (((accs[0] + accs[1]) + (accs[2] + accs[3]))
                 + ((accs[4] + accs[5]) + (accs[6] + accs[7])))

    pltpu.sync_copy(part_v, shared.at[pl.ds(s * _DIM, _DIM)])
    plsc.subcore_barrier()

    @pl.when(s < _CLOUDS_PER_CORE)
    def _():
        pltpu.sync_copy(shared, all_v)
        pltpu.sync_copy(al_hbm, al_v)
        tot = zero
        for k in range(_SUBCORES_PER_CLOUD):
            tot = tot + all_v[pl.ds((s * _SUBCORES_PER_CLOUD + k) * _DIM,
                                    _DIM)]
        for w in range(_N_WEIGHTS):
            prod = al_v[w] * tot
            for j in range(_J + 1):
                row_v[pl.ds((w * (_J + 1) + j) * _DIM, _DIM)] = prod
        pltpu.sync_copy(row_v, out_hbm.at[c * _CLOUDS_PER_CORE + s])


@jax.jit
def _hiponet_pooled(point_clouds, alphas):
    mesh = plsc.VectorSubcoreMesh(core_axis_name="c", subcore_axis_name="s",
                                  num_cores=_NUM_CORES,
                                  num_subcores=_NUM_SUBCORES)
    return pl.kernel(
        _sc_kernel,
        out_type=jax.ShapeDtypeStruct((_B, _OUT_COLS), jnp.float32),
        mesh=mesh,
        scratch_types=[
            pltpu.VMEM((_ROWS_PER_SUBCORE, _DIM), jnp.float32),
            pltpu.VMEM((_DIM,), jnp.float32),
            pltpu.VMEM((_NUM_SUBCORES * _DIM,), jnp.float32),
            pltpu.VMEM((_N_WEIGHTS, _DIM), jnp.float32),
            pltpu.VMEM((_OUT_COLS,), jnp.float32),
            pltpu.VMEM_SHARED((_NUM_SUBCORES * _DIM,), jnp.float32),
        ],
        name="hiponet_pooled_sc",
    )(point_clouds, alphas)


def kernel(point_clouds, alphas, sigma):
    del sigma
    return _hiponet_pooled(point_clouds, alphas)


# speedup vs baseline: 82.5389x; 1.0022x over previous
import functools

import jax
import jax.numpy as jnp
from jax import lax
from jax.experimental import pallas as pl
from jax.experimental.pallas import tpu as pltpu
from jax.experimental.pallas import tpu_sc as plsc

_N_WEIGHTS = 4
_DIM = 16
_J = 3
_B = 4
_N = 4096

_NUM_CORES = 2
_NUM_SUBCORES = 16
_CLOUDS_PER_CORE = _B // _NUM_CORES
_SUBCORES_PER_CLOUD = _NUM_SUBCORES // _CLOUDS_PER_CORE
_ROWS_PER_SUBCORE = _N // _SUBCORES_PER_CLOUD
_OUT_COLS = _N_WEIGHTS * (_J + 1) * _DIM


def _sc_kernel(pc_hbm, al_hbm, out_hbm, rows_v, part_v, all_v, al_v, row_v,
               shared):
    c = lax.axis_index("c")
    s = lax.axis_index("s")
    zero = jnp.zeros((_DIM,), jnp.float32)

    p = c * _CLOUDS_PER_CORE + s // _SUBCORES_PER_CLOUD
    base = (s % _SUBCORES_PER_CLOUD) * _ROWS_PER_SUBCORE
    pltpu.sync_copy(pc_hbm.at[p, pl.ds(base, _ROWS_PER_SUBCORE), :], rows_v)

    def body(i, acc):
        a0, a1, a2, a3, a4, a5, a6, a7 = acc
        a0 = a0 + rows_v[8 * i]
        a1 = a1 + rows_v[8 * i + 1]
        a2 = a2 + rows_v[8 * i + 2]
        a3 = a3 + rows_v[8 * i + 3]
        a4 = a4 + rows_v[8 * i + 4]
        a5 = a5 + rows_v[8 * i + 5]
        a6 = a6 + rows_v[8 * i + 6]
        a7 = a7 + rows_v[8 * i + 7]
        return (a0, a1, a2, a3, a4, a5, a6, a7)

    accs = lax.fori_loop(0, _ROWS_PER_SUBCORE // 8, body, (zero,) * 8)
    part_v[:] = (((accs[0] + accs[1]) + (accs[2] + accs[3]))
                 + ((accs[4] + accs[5]) + (accs[6] + accs[7])))

    pltpu.sync_copy(part_v, shared.at[pl.ds(s * _DIM, _DIM)])
    plsc.subcore_barrier()

    @pl.when(s < _CLOUDS_PER_CORE)
    def _():
        pltpu.sync_copy(shared, all_v)
        pltpu.sync_copy(al_hbm, al_v)
        tot = zero
        for k in range(_SUBCORES_PER_CLOUD):
            tot = tot + all_v[pl.ds((s * _SUBCORES_PER_CLOUD + k) * _DIM,
                                    _DIM)]
        for w in range(_N_WEIGHTS):
            prod = al_v[w] * tot
            for j in range(_J + 1):
                row_v[pl.ds((w * (_J + 1) + j) * _DIM, _DIM)] = prod
        pltpu.sync_copy(row_v, out_hbm.at[c * _CLOUDS_PER_CORE + s])


@jax.jit
def _hiponet_pooled(point_clouds, alphas):
    mesh = plsc.VectorSubcoreMesh(core_axis_name="c", subcore_axis_name="s",
                                  num_cores=_NUM_CORES,
                                  num_subcores=_NUM_SUBCORES)
    return pl.kernel(
        _sc_kernel,
        out_type=jax.ShapeDtypeStruct((_B, _OUT_COLS), jnp.float32),
        mesh=mesh,
        scratch_types=[
            pltpu.VMEM((_ROWS_PER_SUBCORE, _DIM), jnp.float32),
            pltpu.VMEM((_DIM,), jnp.float32),
            pltpu.VMEM((_NUM_SUBCORES * _DIM,), jnp.float32),
            pltpu.VMEM((_N_WEIGHTS, _DIM), jnp.float32),
            pltpu.VMEM((_OUT_COLS,), jnp.float32),
            pltpu.VMEM_SHARED((_NUM_SUBCORES * _DIM,), jnp.float32),
        ],
        compiler_params=pltpu.CompilerParams(use_tc_tiling_on_sc=True),
        name="hiponet_pooled_sc",
    )(point_clouds, alphas)


def kernel(point_clouds, alphas, sigma):
    del sigma
    return _hiponet_pooled(point_clouds, alphas)


# speedup vs baseline: 84.1244x; 1.0192x over previous
import functools

import jax
import jax.numpy as jnp
from jax import lax
from jax.experimental import pallas as pl
from jax.experimental.pallas import tpu as pltpu
from jax.experimental.pallas import tpu_sc as plsc

_N_WEIGHTS = 4
_DIM = 16
_J = 3
_B = 4
_N = 4096

_NUM_CORES = 2
_NUM_SUBCORES = 16
_CLOUDS_PER_CORE = _B // _NUM_CORES
_SUBCORES_PER_CLOUD = _NUM_SUBCORES // _CLOUDS_PER_CORE
_ROWS_PER_SUBCORE = _N // _SUBCORES_PER_CLOUD
_OUT_COLS = _N_WEIGHTS * (_J + 1) * _DIM


def _sc_kernel(pc_hbm, al_hbm, out_hbm, rows_v, part_v, all_v, al_v, row_v,
               shared, sem0, sem1):
    c = lax.axis_index("c")
    s = lax.axis_index("s")
    zero = jnp.zeros((_DIM,), jnp.float32)
    half = _ROWS_PER_SUBCORE // 2

    p = c * _CLOUDS_PER_CORE + s // _SUBCORES_PER_CLOUD
    base = (s % _SUBCORES_PER_CLOUD) * _ROWS_PER_SUBCORE
    cp0 = pltpu.async_copy(pc_hbm.at[p, pl.ds(base, half), :],
                           rows_v.at[pl.ds(0, half), :], sem0)
    cp1 = pltpu.async_copy(pc_hbm.at[p, pl.ds(base + half, half), :],
                           rows_v.at[pl.ds(half, half), :], sem1)

    @pl.when(s < _CLOUDS_PER_CORE)
    def _():
        pltpu.sync_copy(al_hbm, al_v)

    def body(i, acc):
        a0, a1, a2, a3, a4, a5, a6, a7 = acc
        a0 = a0 + rows_v[8 * i]
        a1 = a1 + rows_v[8 * i + 1]
        a2 = a2 + rows_v[8 * i + 2]
        a3 = a3 + rows_v[8 * i + 3]
        a4 = a4 + rows_v[8 * i + 4]
        a5 = a5 + rows_v[8 * i + 5]
        a6 = a6 + rows_v[8 * i + 6]
        a7 = a7 + rows_v[8 * i + 7]
        return (a0, a1, a2, a3, a4, a5, a6, a7)

    cp0.wait()
    accs = lax.fori_loop(0, half // 8, body, (zero,) * 8)
    cp1.wait()
    accs = lax.fori_loop(half // 8, _ROWS_PER_SUBCORE // 8, body, accs)
    part_v[:] = (((accs[0] + accs[1]) + (accs[2] + accs[3]))
                 + ((accs[4] + accs[5]) + (accs[6] + accs[7])))

    pltpu.sync_copy(part_v, shared.at[pl.ds(s * _DIM, _DIM)])
    plsc.subcore_barrier()

    @pl.when(s < _CLOUDS_PER_CORE)
    def _():
        pltpu.sync_copy(shared, all_v)
        tot = zero
        for k in range(_SUBCORES_PER_CLOUD):
            tot = tot + all_v[pl.ds((s * _SUBCORES_PER_CLOUD + k) * _DIM,
                                    _DIM)]
        for w in range(_N_WEIGHTS):
            prod = al_v[w] * tot
            for j in range(_J + 1):
                row_v[pl.ds((w * (_J + 1) + j) * _DIM, _DIM)] = prod
        pltpu.sync_copy(row_v, out_hbm.at[c * _CLOUDS_PER_CORE + s])


@jax.jit
def _hiponet_pooled(point_clouds, alphas):
    mesh = plsc.VectorSubcoreMesh(core_axis_name="c", subcore_axis_name="s",
                                  num_cores=_NUM_CORES,
                                  num_subcores=_NUM_SUBCORES)
    return pl.kernel(
        _sc_kernel,
        out_type=jax.ShapeDtypeStruct((_B, _OUT_COLS), jnp.float32),
        mesh=mesh,
        scratch_types=[
            pltpu.VMEM((_ROWS_PER_SUBCORE, _DIM), jnp.float32),
            pltpu.VMEM((_DIM,), jnp.float32),
            pltpu.VMEM((_NUM_SUBCORES * _DIM,), jnp.float32),
            pltpu.VMEM((_N_WEIGHTS, _DIM), jnp.float32),
            pltpu.VMEM((_OUT_COLS,), jnp.float32),
            pltpu.VMEM_SHARED((_NUM_SUBCORES * _DIM,), jnp.float32),
            pltpu.SemaphoreType.DMA,
            pltpu.SemaphoreType.DMA,
        ],
        name="hiponet_pooled_sc",
    )(point_clouds, alphas)


def kernel(point_clouds, alphas, sigma):
    del sigma
    return _hiponet_pooled(point_clouds, alphas)
